# baseline (speedup 1.0000x reference)
import functools

import jax
import jax.numpy as jnp
from jax import lax
from jax.experimental import pallas as pl
from jax.experimental.pallas import tpu as pltpu
from jax.experimental.pallas import tpu_sc as plsc

N, D, H, O, E, K = 2048, 768, 768, 768, 8, 2
BM = 256
BMM = 512
T = (N * K) // BMM
S = T + E - 1
NW = 32
SLOTS = N * K
GWL = 16


def _gating_body(x_ref, wg_ref, bg_ref, ww_ref, i_ref):
    xv = x_ref[...]
    logits = jnp.dot(xv, wg_ref[...],
                     preferred_element_type=jnp.float32) + bg_ref[...]
    cols = lax.broadcasted_iota(jnp.int32, logits.shape, 1)
    m1 = jnp.max(logits, axis=-1, keepdims=True)
    i1 = jnp.min(jnp.where(logits == m1, cols, E), axis=-1, keepdims=True)
    l2 = jnp.where(cols == i1, -jnp.inf, logits)
    m2 = jnp.max(l2, axis=-1, keepdims=True)
    i2 = jnp.min(jnp.where(l2 == m2, cols, E), axis=-1, keepdims=True)
    e2 = jnp.exp(m2 - m1)
    denom = 1.0 + e2
    ww_ref[...] = jnp.concatenate([1.0 / denom, e2 / denom], axis=-1)
    i_ref[...] = jnp.concatenate([i1, i2], axis=-1)


def _gating(x, Wg, bg):
    return pl.pallas_call(
        _gating_body,
        grid=(N // BM,),
        in_specs=[
            pl.BlockSpec((BM, D), lambda i: (i, 0)),
            pl.BlockSpec((D, E), lambda i: (0, 0)),
            pl.BlockSpec((1, E), lambda i: (0, 0)),
        ],
        out_specs=[
            pl.BlockSpec((BM, K), lambda i: (i, 0)),
            pl.BlockSpec((BM, K), lambda i: (i, 0)),
        ],
        out_shape=[
            jax.ShapeDtypeStruct((N, K), jnp.float32),
            jax.ShapeDtypeStruct((N, K), jnp.int32),
        ],
    )(x, Wg, bg.reshape(1, E))


_RROWS = SLOTS // 128


def _rank_body(e_ref, pos_ref, t_ref, eo_ref, lo_ref, hi_ref):
    f32 = jnp.float32
    ef = e_ref[...]
    lane_tri = (lax.broadcasted_iota(jnp.int32, (128, 128), 0)
                <= lax.broadcasted_iota(jnp.int32, (128, 128), 1)
                ).astype(f32)
    row_strict = (lax.broadcasted_iota(jnp.int32, (_RROWS, _RROWS), 1)
                  < lax.broadcasted_iota(jnp.int32, (_RROWS, _RROWS), 0)
                  ).astype(f32)

    pos = jnp.zeros((_RROWS, 128), f32)
    off = 0.0
    offs_e, offs_i = [], []
    for e in range(E):
        m = (ef == e).astype(f32)
        lane_pref = jnp.dot(m, lane_tri, preferred_element_type=f32)
        rt = jnp.sum(m, axis=1, keepdims=True)
        row_pe = jnp.dot(row_strict, rt, preferred_element_type=f32)
        rank0 = lane_pref + row_pe - 1.0
        offs_e.append(off)
        pos = pos + m * (off + rank0)
        off = off + jnp.sum(rt)
        offs_i.append(off)
    pos_ref[...] = pos.astype(jnp.int32)

    off_e_row = jnp.stack(offs_e).reshape(1, E)
    off_i_row = jnp.stack(offs_i).reshape(1, E)
    off_e_col = jnp.stack(offs_e).reshape(E, 1)
    off_i_col = jnp.stack(offs_i).reshape(E, 1)

    tcol = lax.broadcasted_iota(jnp.int32, (T, 1), 0).astype(f32)
    incl_a = (jnp.minimum((tcol + 1.0) * BMM, off_i_row)
              > jnp.maximum(tcol * BMM, off_e_row)).astype(f32)
    nsteps = jnp.sum(incl_a, axis=1, keepdims=True)
    l16s = (lax.broadcasted_iota(jnp.int32, (T, T), 1)
            < lax.broadcasted_iota(jnp.int32, (T, T), 0)).astype(f32)
    csteps = jnp.dot(l16s, nsteps, preferred_element_type=f32)

    s_row = lax.broadcasted_iota(jnp.int32, (1, S), 1).astype(f32)
    t_s = jnp.sum((csteps <= s_row).astype(f32), axis=0, keepdims=True) - 1.0
    onehot_t = (lax.broadcasted_iota(jnp.int32, (T, S), 0).astype(f32) == t_s).astype(f32)
    cst_s = jnp.sum(onehot_t * csteps, axis=0, keepdims=True)
    nst_s = jnp.sum(onehot_t * nsteps, axis=0, keepdims=True)
    j = jnp.minimum(s_row - cst_s, jnp.maximum(nst_s - 1.0, 0.0))

    trow = lax.broadcasted_iota(jnp.int32, (1, T), 1).astype(f32)
    incl_b = (jnp.minimum((trow + 1.0) * BMM, off_i_col)
              > jnp.maximum(trow * BMM, off_e_col)).astype(f32)
    l8i = (lax.broadcasted_iota(jnp.int32, (E, E), 1)
           <= lax.broadcasted_iota(jnp.int32, (E, E), 0)).astype(f32)
    cc_b = jnp.dot(l8i, incl_b, preferred_element_type=f32)
    cc_sel = jnp.dot(cc_b, onehot_t, preferred_element_type=f32)
    e_s = jnp.sum((cc_sel <= j).astype(f32), axis=0, keepdims=True)
    e_s = jnp.minimum(e_s, float(E - 1))
    onehot_e = (lax.broadcasted_iota(jnp.int32, (E, S), 0).astype(f32) == e_s).astype(f32)
    osel_e = jnp.sum(onehot_e * off_e_col, axis=0, keepdims=True)
    osel_i = jnp.sum(onehot_e * off_i_col, axis=0, keepdims=True)
    lo_s = jnp.maximum(t_s * BMM, osel_e)
    hi_s = jnp.minimum((t_s + 1.0) * BMM, osel_i)
    total_steps = jnp.sum(nsteps)
    padm = s_row >= total_steps
    lo_s = jnp.where(padm, 0.0, lo_s)
    hi_s = jnp.where(padm, 0.0, hi_s)

    t_ref[...] = t_s.astype(jnp.int32)
    eo_ref[...] = e_s.astype(jnp.int32)
    lo_ref[...] = lo_s.astype(jnp.int32)
    hi_ref[...] = hi_s.astype(jnp.int32)


def _rank(e2d):
    return pl.pallas_call(
        _rank_body,
        out_shape=[
            jax.ShapeDtypeStruct((_RROWS, 128), jnp.int32),
            jax.ShapeDtypeStruct((1, S), jnp.int32),
            jax.ShapeDtypeStruct((1, S), jnp.int32),
            jax.ShapeDtypeStruct((1, S), jnp.int32),
            jax.ShapeDtypeStruct((1, S), jnp.int32),
        ],
    )(e2d)


def _sc_dispatch(x, pos0, pos1):
    ptok = N // NW
    pslot = SLOTS // NW
    mesh = plsc.VectorSubcoreMesh(core_axis_name="c", subcore_axis_name="s")

    @functools.partial(
        pl.kernel, mesh=mesh,
        out_type=jax.ShapeDtypeStruct((SLOTS, D), jnp.float32),
        scratch_types=[
            pltpu.VMEM((ptok, D), jnp.float32),
            pltpu.VMEM((ptok,), jnp.int32),
            pltpu.VMEM((ptok,), jnp.int32),
        ],
    )
    def k(x_hbm, p0_hbm, p1_hbm, xs_hbm, xv, pv0, pv1):
        wid = lax.axis_index("s") * 2 + lax.axis_index("c")
        bt = wid * ptok
        pltpu.sync_copy(p0_hbm.at[pl.ds(bt, ptok)], pv0)
        pltpu.sync_copy(p1_hbm.at[pl.ds(bt, ptok)], pv1)
        pltpu.sync_copy(x_hbm.at[pl.ds(bt, ptok)], xv)
        pltpu.sync_copy(xv, xs_hbm.at[pv0])
        pltpu.sync_copy(xv, xs_hbm.at[pv1])

    return k(x, pos0, pos1)


def _mlp_body(tile_s, exp_s, lo_s, hi_s,
              xs_ref, w1_ref, b1_ref, w2_ref, b2_ref, out_ref):
    s = pl.program_id(0)
    lo, hi = lo_s[s], hi_s[s]

    @pl.when(hi > lo)
    def _():
        xb = xs_ref[...].astype(jnp.bfloat16)
        h = jnp.dot(xb, w1_ref[0], preferred_element_type=jnp.float32)
        h = jnp.maximum(h + b1_ref[0], 0.0).astype(jnp.bfloat16)
        y = jnp.dot(h, w2_ref[0], preferred_element_type=jnp.float32)
        y = y + b2_ref[0]
        base = tile_s[s] * BMM
        rows = base + lax.broadcasted_iota(jnp.int32, (BMM, 1), 0)
        mask = (rows >= lo) & (rows < hi)
        out_ref[...] = jnp.where(mask, y, out_ref[...])


def _mlp_grouped(xs, W1, b1, W2, b2, tile_s, exp_s, lo_s, hi_s):
    grid_spec = pltpu.PrefetchScalarGridSpec(
        num_scalar_prefetch=4,
        grid=(S,),
        in_specs=[
            pl.BlockSpec((BMM, D), lambda s, t, e, lo, hi: (t[s], 0)),
            pl.BlockSpec((1, D, H), lambda s, t, e, lo, hi: (e[s], 0, 0)),
            pl.BlockSpec((1, 1, H), lambda s, t, e, lo, hi: (e[s], 0, 0)),
            pl.BlockSpec((1, H, O), lambda s, t, e, lo, hi: (e[s], 0, 0)),
            pl.BlockSpec((1, 1, O), lambda s, t, e, lo, hi: (e[s], 0, 0)),
        ],
        out_specs=pl.BlockSpec((BMM, O), lambda s, t, e, lo, hi: (t[s], 0)),
    )
    return pl.pallas_call(
        _mlp_body,
        grid_spec=grid_spec,
        out_shape=jax.ShapeDtypeStruct((SLOTS, O), jnp.float32),
    )(tile_s, exp_s, lo_s, hi_s, xs, W1.astype(jnp.bfloat16),
      b1.reshape(E, 1, H), W2.astype(jnp.bfloat16), b2.reshape(E, 1, O))


def _sc_combine_gather(ys, posI):
    pslot = SLOTS // NW
    mesh = plsc.VectorSubcoreMesh(core_axis_name="c", subcore_axis_name="s")

    @functools.partial(
        pl.kernel, mesh=mesh,
        out_type=jax.ShapeDtypeStruct((SLOTS, O), jnp.float32),
        scratch_types=[
            pltpu.VMEM((pslot,), jnp.int32),
            pltpu.VMEM((pslot, O), jnp.float32),
            pltpu.SemaphoreType.DMA,
        ],
    )
    def k(ys_hbm, pi_hbm, g_hbm, pvi, rows_v, sem):
        wid = lax.axis_index("s") * 2 + lax.axis_index("c")
        bs = wid * pslot
        pltpu.sync_copy(pi_hbm.at[pl.ds(bs, pslot)], pvi)
        pltpu.async_copy(ys_hbm.at[pvi], rows_v, sem).wait()
        pltpu.sync_copy(rows_v, g_hbm.at[pl.ds(bs, pslot)])

    return k(ys, posI)


def _add_body(g_ref, w_ref, o_ref):
    o_ref[...] = (w_ref[:, 0:1] * g_ref[:, :O]
                  + w_ref[:, 1:2] * g_ref[:, O:])


def _pair_add(g2, gate_w):
    return pl.pallas_call(
        _add_body,
        grid=(N // BM,),
        in_specs=[pl.BlockSpec((BM, K * O), lambda i: (i, 0)),
                  pl.BlockSpec((BM, K), lambda i: (i, 0))],
        out_specs=pl.BlockSpec((BM, O), lambda i: (i, 0)),
        out_shape=jax.ShapeDtypeStruct((N, O), jnp.float32),
    )(g2, gate_w)


def kernel(x, Wg, bg, W1, b1, W2, b2):
    gate_w, gate_i = _gating(x, Wg, bg)
    pos2d, t_s, e_s, lo_s, hi_s = _rank(gate_i.reshape(_RROWS, 128))
    posI = pos2d.reshape(SLOTS)
    posnk = pos2d.reshape(N, K)
    xs = _sc_dispatch(x, posnk[:, 0], posnk[:, 1])
    ys = _mlp_grouped(xs, W1, b1, W2, b2,
                      t_s.reshape(S), e_s.reshape(S),
                      lo_s.reshape(S), hi_s.reshape(S))
    g = _sc_combine_gather(ys, posI)
    return _pair_add(g.reshape(N, K * O), gate_w)
    posI = pos2d.reshape(SLOTS)
    posnk = pos2d.reshape(N, K)

    xs = _sc_dispatch(x, posnk[:, 0], posnk[:, 1])
    ys = _mlp_grouped(xs, W1, b1, W2, b2,
                      t_s.reshape(S), e_s.reshape(S),
                      lo_s.reshape(S), hi_s.reshape(S))

    g = _sc_combine_gather(ys, posI)
    return _pair_add(g.reshape(N, K, O), gate_w)


# speedup vs baseline: 1.1006x; 1.1006x over previous
import functools

import jax
import jax.numpy as jnp
from jax import lax
from jax.experimental import pallas as pl
from jax.experimental.pallas import tpu as pltpu
from jax.experimental.pallas import tpu_sc as plsc

N, D, H, O, E, K = 2048, 768, 768, 768, 8, 2
BM = 256
BMM = 512
T = (N * K) // BMM
S = T + E - 1
NW = 32
SLOTS = N * K
GWL = 16


def _gating_body(x_ref, wg_ref, bg_ref, ww_ref, i_ref):
    xv = x_ref[...]
    logits = jnp.dot(xv, wg_ref[...],
                     preferred_element_type=jnp.float32) + bg_ref[...]
    cols = lax.broadcasted_iota(jnp.int32, logits.shape, 1)
    m1 = jnp.max(logits, axis=-1, keepdims=True)
    i1 = jnp.min(jnp.where(logits == m1, cols, E), axis=-1, keepdims=True)
    l2 = jnp.where(cols == i1, -jnp.inf, logits)
    m2 = jnp.max(l2, axis=-1, keepdims=True)
    i2 = jnp.min(jnp.where(l2 == m2, cols, E), axis=-1, keepdims=True)
    e2 = jnp.exp(m2 - m1)
    denom = 1.0 + e2
    ww_ref[...] = jnp.concatenate([1.0 / denom, e2 / denom], axis=-1)
    i_ref[...] = jnp.concatenate([i1, i2], axis=-1)


def _gating(x, Wg, bg):
    return pl.pallas_call(
        _gating_body,
        grid=(N // BM,),
        in_specs=[
            pl.BlockSpec((BM, D), lambda i: (i, 0)),
            pl.BlockSpec((D, E), lambda i: (0, 0)),
            pl.BlockSpec((1, E), lambda i: (0, 0)),
        ],
        out_specs=[
            pl.BlockSpec((BM, K), lambda i: (i, 0)),
            pl.BlockSpec((BM, K), lambda i: (i, 0)),
        ],
        out_shape=[
            jax.ShapeDtypeStruct((N, K), jnp.float32),
            jax.ShapeDtypeStruct((N, K), jnp.int32),
        ],
    )(x, Wg, bg.reshape(1, E))


_RROWS = SLOTS // 128


def _rank_body(e_ref, pos_ref, t_ref, eo_ref, lo_ref, hi_ref):
    f32 = jnp.float32
    ef = e_ref[...]
    lane_tri = (lax.broadcasted_iota(jnp.int32, (128, 128), 0)
                <= lax.broadcasted_iota(jnp.int32, (128, 128), 1)
                ).astype(f32)
    row_strict = (lax.broadcasted_iota(jnp.int32, (_RROWS, _RROWS), 1)
                  < lax.broadcasted_iota(jnp.int32, (_RROWS, _RROWS), 0)
                  ).astype(f32)

    pos = jnp.zeros((_RROWS, 128), f32)
    off = 0.0
    offs_e, offs_i = [], []
    for e in range(E):
        m = (ef == e).astype(f32)
        lane_pref = jnp.dot(m, lane_tri, preferred_element_type=f32)
        rt = jnp.sum(m, axis=1, keepdims=True)
        row_pe = jnp.dot(row_strict, rt, preferred_element_type=f32)
        rank0 = lane_pref + row_pe - 1.0
        offs_e.append(off)
        pos = pos + m * (off + rank0)
        off = off + jnp.sum(rt)
        offs_i.append(off)
    pos_ref[...] = pos.astype(jnp.int32)

    off_e_row = jnp.stack(offs_e).reshape(1, E)
    off_i_row = jnp.stack(offs_i).reshape(1, E)
    off_e_col = jnp.stack(offs_e).reshape(E, 1)
    off_i_col = jnp.stack(offs_i).reshape(E, 1)

    tcol = lax.broadcasted_iota(jnp.int32, (T, 1), 0).astype(f32)
    incl_a = (jnp.minimum((tcol + 1.0) * BMM, off_i_row)
              > jnp.maximum(tcol * BMM, off_e_row)).astype(f32)
    nsteps = jnp.sum(incl_a, axis=1, keepdims=True)
    l16s = (lax.broadcasted_iota(jnp.int32, (T, T), 1)
            < lax.broadcasted_iota(jnp.int32, (T, T), 0)).astype(f32)
    csteps = jnp.dot(l16s, nsteps, preferred_element_type=f32)

    s_row = lax.broadcasted_iota(jnp.int32, (1, S), 1).astype(f32)
    t_s = jnp.sum((csteps <= s_row).astype(f32), axis=0, keepdims=True) - 1.0
    onehot_t = (lax.broadcasted_iota(jnp.int32, (T, S), 0).astype(f32) == t_s).astype(f32)
    cst_s = jnp.sum(onehot_t * csteps, axis=0, keepdims=True)
    nst_s = jnp.sum(onehot_t * nsteps, axis=0, keepdims=True)
    j = jnp.minimum(s_row - cst_s, jnp.maximum(nst_s - 1.0, 0.0))

    trow = lax.broadcasted_iota(jnp.int32, (1, T), 1).astype(f32)
    incl_b = (jnp.minimum((trow + 1.0) * BMM, off_i_col)
              > jnp.maximum(trow * BMM, off_e_col)).astype(f32)
    l8i = (lax.broadcasted_iota(jnp.int32, (E, E), 1)
           <= lax.broadcasted_iota(jnp.int32, (E, E), 0)).astype(f32)
    cc_b = jnp.dot(l8i, incl_b, preferred_element_type=f32)
    cc_sel = jnp.dot(cc_b, onehot_t, preferred_element_type=f32)
    e_s = jnp.sum((cc_sel <= j).astype(f32), axis=0, keepdims=True)
    e_s = jnp.minimum(e_s, float(E - 1))
    onehot_e = (lax.broadcasted_iota(jnp.int32, (E, S), 0).astype(f32) == e_s).astype(f32)
    osel_e = jnp.sum(onehot_e * off_e_col, axis=0, keepdims=True)
    osel_i = jnp.sum(onehot_e * off_i_col, axis=0, keepdims=True)
    lo_s = jnp.maximum(t_s * BMM, osel_e)
    hi_s = jnp.minimum((t_s + 1.0) * BMM, osel_i)
    total_steps = jnp.sum(nsteps)
    padm = s_row >= total_steps
    lo_s = jnp.where(padm, 0.0, lo_s)
    hi_s = jnp.where(padm, 0.0, hi_s)

    t_ref[...] = t_s.astype(jnp.int32)
    eo_ref[...] = e_s.astype(jnp.int32)
    lo_ref[...] = lo_s.astype(jnp.int32)
    hi_ref[...] = hi_s.astype(jnp.int32)


def _rank(e2d):
    return pl.pallas_call(
        _rank_body,
        out_shape=[
            jax.ShapeDtypeStruct((_RROWS, 128), jnp.int32),
            jax.ShapeDtypeStruct((1, S), jnp.int32),
            jax.ShapeDtypeStruct((1, S), jnp.int32),
            jax.ShapeDtypeStruct((1, S), jnp.int32),
            jax.ShapeDtypeStruct((1, S), jnp.int32),
        ],
    )(e2d)


def _sc_dispatch(x, pos0, pos1):
    ptok = N // NW
    pslot = SLOTS // NW
    mesh = plsc.VectorSubcoreMesh(core_axis_name="c", subcore_axis_name="s")

    @functools.partial(
        pl.kernel, mesh=mesh,
        out_type=jax.ShapeDtypeStruct((SLOTS, D), jnp.float32),
        scratch_types=[
            pltpu.VMEM((ptok, D), jnp.float32),
            pltpu.VMEM((ptok,), jnp.int32),
            pltpu.VMEM((ptok,), jnp.int32),
        ],
    )
    def k(x_hbm, p0_hbm, p1_hbm, xs_hbm, xv, pv0, pv1):
        wid = lax.axis_index("s") * 2 + lax.axis_index("c")
        bt = wid * ptok
        pltpu.sync_copy(p0_hbm.at[pl.ds(bt, ptok)], pv0)
        pltpu.sync_copy(p1_hbm.at[pl.ds(bt, ptok)], pv1)
        pltpu.sync_copy(x_hbm.at[pl.ds(bt, ptok)], xv)
        pltpu.sync_copy(xv, xs_hbm.at[pv0])
        pltpu.sync_copy(xv, xs_hbm.at[pv1])

    return k(x, pos0, pos1)


def _mlp_body(tile_s, exp_s, lo_s, hi_s,
              xs_ref, w1_ref, b1_ref, w2_ref, b2_ref, out_ref):
    s = pl.program_id(0)
    lo, hi = lo_s[s], hi_s[s]

    @pl.when(hi > lo)
    def _():
        xb = xs_ref[...].astype(jnp.bfloat16)
        h = jnp.dot(xb, w1_ref[0].astype(jnp.bfloat16),
                    preferred_element_type=jnp.float32)
        h = jnp.maximum(h + b1_ref[0], 0.0).astype(jnp.bfloat16)
        y = jnp.dot(h, w2_ref[0].astype(jnp.bfloat16),
                    preferred_element_type=jnp.float32)
        y = y + b2_ref[0]
        base = tile_s[s] * BMM
        rows = base + lax.broadcasted_iota(jnp.int32, (BMM, 1), 0)
        mask = (rows >= lo) & (rows < hi)
        out_ref[...] = jnp.where(mask, y, out_ref[...])


def _mlp_grouped(xs, W1, b1, W2, b2, tile_s, exp_s, lo_s, hi_s):
    grid_spec = pltpu.PrefetchScalarGridSpec(
        num_scalar_prefetch=4,
        grid=(S,),
        in_specs=[
            pl.BlockSpec((BMM, D), lambda s, t, e, lo, hi: (t[s], 0)),
            pl.BlockSpec((1, D, H), lambda s, t, e, lo, hi: (e[s], 0, 0)),
            pl.BlockSpec((1, 1, H), lambda s, t, e, lo, hi: (e[s], 0, 0)),
            pl.BlockSpec((1, H, O), lambda s, t, e, lo, hi: (e[s], 0, 0)),
            pl.BlockSpec((1, 1, O), lambda s, t, e, lo, hi: (e[s], 0, 0)),
        ],
        out_specs=pl.BlockSpec((BMM, O), lambda s, t, e, lo, hi: (t[s], 0)),
    )
    return pl.pallas_call(
        _mlp_body,
        grid_spec=grid_spec,
        out_shape=jax.ShapeDtypeStruct((SLOTS, O), jnp.float32),
    )(tile_s, exp_s, lo_s, hi_s, xs, W1, b1.reshape(E, 1, H), W2,
      b2.reshape(E, 1, O))


def _sc_combine_gather(ys, posI):
    pslot = SLOTS // NW
    mesh = plsc.VectorSubcoreMesh(core_axis_name="c", subcore_axis_name="s")

    @functools.partial(
        pl.kernel, mesh=mesh,
        out_type=jax.ShapeDtypeStruct((SLOTS, O), jnp.float32),
        scratch_types=[
            pltpu.VMEM((pslot,), jnp.int32),
            pltpu.VMEM((pslot, O), jnp.float32),
            pltpu.SemaphoreType.DMA,
        ],
    )
    def k(ys_hbm, pi_hbm, g_hbm, pvi, rows_v, sem):
        wid = lax.axis_index("s") * 2 + lax.axis_index("c")
        bs = wid * pslot
        pltpu.sync_copy(pi_hbm.at[pl.ds(bs, pslot)], pvi)
        pltpu.async_copy(ys_hbm.at[pvi], rows_v, sem).wait()
        pltpu.sync_copy(rows_v, g_hbm.at[pl.ds(bs, pslot)])

    return k(ys, posI)


def _add_body(g_ref, w_ref, o_ref):
    o_ref[...] = (w_ref[:, 0:1] * g_ref[:, :O]
                  + w_ref[:, 1:2] * g_ref[:, O:])


def _pair_add(g2, gate_w):
    return pl.pallas_call(
        _add_body,
        grid=(N // BM,),
        in_specs=[pl.BlockSpec((BM, K * O), lambda i: (i, 0)),
                  pl.BlockSpec((BM, K), lambda i: (i, 0))],
        out_specs=pl.BlockSpec((BM, O), lambda i: (i, 0)),
        out_shape=jax.ShapeDtypeStruct((N, O), jnp.float32),
    )(g2, gate_w)


def kernel(x, Wg, bg, W1, b1, W2, b2):
    gate_w, gate_i = _gating(x, Wg, bg)
    pos2d, t_s, e_s, lo_s, hi_s = _rank(gate_i.reshape(_RROWS, 128))
    posI = pos2d.reshape(SLOTS)
    posnk = pos2d.reshape(N, K)
    xs = _sc_dispatch(x, posnk[:, 0], posnk[:, 1])
    ys = _mlp_grouped(xs, W1, b1, W2, b2,
                      t_s.reshape(S), e_s.reshape(S),
                      lo_s.reshape(S), hi_s.reshape(S))
    g = _sc_combine_gather(ys, posI)
    return _pair_add(g.reshape(N, K * O), gate_w)
    posI = pos2d.reshape(SLOTS)
    posnk = pos2d.reshape(N, K)

    xs = _sc_dispatch(x, posnk[:, 0], posnk[:, 1])
    ys = _mlp_grouped(xs, W1, b1, W2, b2,
                      t_s.reshape(S), e_s.reshape(S),
                      lo_s.reshape(S), hi_s.reshape(S))

    g = _sc_combine_gather(ys, posI)
    return _pair_add(g.reshape(N, K, O), gate_w)


# speedup vs baseline: 1.1089x; 1.0076x over previous
import functools

import jax
import jax.numpy as jnp
from jax import lax
from jax.experimental import pallas as pl
from jax.experimental.pallas import tpu as pltpu
from jax.experimental.pallas import tpu_sc as plsc

N, D, H, O, E, K = 2048, 768, 768, 768, 8, 2
BM = 256
BMM = 512
T = (N * K) // BMM
S = T + E - 1
NW = 32
SLOTS = N * K
GWL = 16


def _gating_body(x_ref, wg_ref, bg_ref, ww_ref, i_ref):
    xv = x_ref[...]
    logits = jnp.dot(xv, wg_ref[...],
                     preferred_element_type=jnp.float32) + bg_ref[...]
    cols = lax.broadcasted_iota(jnp.int32, logits.shape, 1)
    m1 = jnp.max(logits, axis=-1, keepdims=True)
    i1 = jnp.min(jnp.where(logits == m1, cols, E), axis=-1, keepdims=True)
    l2 = jnp.where(cols == i1, -jnp.inf, logits)
    m2 = jnp.max(l2, axis=-1, keepdims=True)
    i2 = jnp.min(jnp.where(l2 == m2, cols, E), axis=-1, keepdims=True)
    e2 = jnp.exp(m2 - m1)
    denom = 1.0 + e2
    ww_ref[...] = jnp.concatenate([1.0 / denom, e2 / denom], axis=-1)
    i_ref[...] = jnp.concatenate([i1, i2], axis=-1)


def _gating(x, Wg, bg):
    return pl.pallas_call(
        _gating_body,
        grid=(N // BM,),
        in_specs=[
            pl.BlockSpec((BM, D), lambda i: (i, 0)),
            pl.BlockSpec((D, E), lambda i: (0, 0)),
            pl.BlockSpec((1, E), lambda i: (0, 0)),
        ],
        out_specs=[
            pl.BlockSpec((BM, K), lambda i: (i, 0)),
            pl.BlockSpec((BM, K), lambda i: (i, 0)),
        ],
        out_shape=[
            jax.ShapeDtypeStruct((N, K), jnp.float32),
            jax.ShapeDtypeStruct((N, K), jnp.int32),
        ],
    )(x, Wg, bg.reshape(1, E))


_RROWS = SLOTS // 128


def _rank_body(e_ref, pos_ref, t_ref, eo_ref, lo_ref, hi_ref):
    f32 = jnp.float32
    ef = e_ref[...]
    lane_tri = (lax.broadcasted_iota(jnp.int32, (128, 128), 0)
                <= lax.broadcasted_iota(jnp.int32, (128, 128), 1)
                ).astype(f32)
    row_strict = (lax.broadcasted_iota(jnp.int32, (_RROWS, _RROWS), 1)
                  < lax.broadcasted_iota(jnp.int32, (_RROWS, _RROWS), 0)
                  ).astype(f32)

    pos = jnp.zeros((_RROWS, 128), f32)
    off = 0.0
    offs_e, offs_i = [], []
    for e in range(E):
        m = (ef == e).astype(f32)
        lane_pref = jnp.dot(m, lane_tri, preferred_element_type=f32)
        rt = jnp.sum(m, axis=1, keepdims=True)
        row_pe = jnp.dot(row_strict, rt, preferred_element_type=f32)
        rank0 = lane_pref + row_pe - 1.0
        offs_e.append(off)
        pos = pos + m * (off + rank0)
        off = off + jnp.sum(rt)
        offs_i.append(off)
    pos_ref[...] = pos.astype(jnp.int32)

    off_e_row = jnp.stack(offs_e).reshape(1, E)
    off_i_row = jnp.stack(offs_i).reshape(1, E)
    off_e_col = jnp.stack(offs_e).reshape(E, 1)
    off_i_col = jnp.stack(offs_i).reshape(E, 1)

    tcol = lax.broadcasted_iota(jnp.int32, (T, 1), 0).astype(f32)
    incl_a = (jnp.minimum((tcol + 1.0) * BMM, off_i_row)
              > jnp.maximum(tcol * BMM, off_e_row)).astype(f32)
    nsteps = jnp.sum(incl_a, axis=1, keepdims=True)
    l16s = (lax.broadcasted_iota(jnp.int32, (T, T), 1)
            < lax.broadcasted_iota(jnp.int32, (T, T), 0)).astype(f32)
    csteps = jnp.dot(l16s, nsteps, preferred_element_type=f32)

    s_row = lax.broadcasted_iota(jnp.int32, (1, S), 1).astype(f32)
    t_s = jnp.sum((csteps <= s_row).astype(f32), axis=0, keepdims=True) - 1.0
    onehot_t = (lax.broadcasted_iota(jnp.int32, (T, S), 0).astype(f32) == t_s).astype(f32)
    cst_s = jnp.sum(onehot_t * csteps, axis=0, keepdims=True)
    nst_s = jnp.sum(onehot_t * nsteps, axis=0, keepdims=True)
    j = jnp.minimum(s_row - cst_s, jnp.maximum(nst_s - 1.0, 0.0))

    trow = lax.broadcasted_iota(jnp.int32, (1, T), 1).astype(f32)
    incl_b = (jnp.minimum((trow + 1.0) * BMM, off_i_col)
              > jnp.maximum(trow * BMM, off_e_col)).astype(f32)
    l8i = (lax.broadcasted_iota(jnp.int32, (E, E), 1)
           <= lax.broadcasted_iota(jnp.int32, (E, E), 0)).astype(f32)
    cc_b = jnp.dot(l8i, incl_b, preferred_element_type=f32)
    cc_sel = jnp.dot(cc_b, onehot_t, preferred_element_type=f32)
    e_s = jnp.sum((cc_sel <= j).astype(f32), axis=0, keepdims=True)
    e_s = jnp.minimum(e_s, float(E - 1))
    onehot_e = (lax.broadcasted_iota(jnp.int32, (E, S), 0).astype(f32) == e_s).astype(f32)
    osel_e = jnp.sum(onehot_e * off_e_col, axis=0, keepdims=True)
    osel_i = jnp.sum(onehot_e * off_i_col, axis=0, keepdims=True)
    lo_s = jnp.maximum(t_s * BMM, osel_e)
    hi_s = jnp.minimum((t_s + 1.0) * BMM, osel_i)
    total_steps = jnp.sum(nsteps)
    padm = s_row >= total_steps
    lo_s = jnp.where(padm, 0.0, lo_s)
    hi_s = jnp.where(padm, 0.0, hi_s)

    t_ref[...] = t_s.astype(jnp.int32)
    eo_ref[...] = e_s.astype(jnp.int32)
    lo_ref[...] = lo_s.astype(jnp.int32)
    hi_ref[...] = hi_s.astype(jnp.int32)


def _rank(e2d):
    return pl.pallas_call(
        _rank_body,
        out_shape=[
            jax.ShapeDtypeStruct((_RROWS, 128), jnp.int32),
            jax.ShapeDtypeStruct((1, S), jnp.int32),
            jax.ShapeDtypeStruct((1, S), jnp.int32),
            jax.ShapeDtypeStruct((1, S), jnp.int32),
            jax.ShapeDtypeStruct((1, S), jnp.int32),
        ],
    )(e2d)


def _sc_dispatch(x, pos0, pos1):
    ptok = N // NW
    pslot = SLOTS // NW
    mesh = plsc.VectorSubcoreMesh(core_axis_name="c", subcore_axis_name="s")

    @functools.partial(
        pl.kernel, mesh=mesh,
        out_type=jax.ShapeDtypeStruct((SLOTS, D), jnp.float32),
        scratch_types=[
            pltpu.VMEM((ptok, D), jnp.float32),
            pltpu.VMEM((ptok,), jnp.int32),
            pltpu.VMEM((ptok,), jnp.int32),
            pltpu.SemaphoreType.DMA,
            pltpu.SemaphoreType.DMA,
            pltpu.SemaphoreType.DMA,
        ],
    )
    def k(x_hbm, p0_hbm, p1_hbm, xs_hbm, xv, pv0, pv1, s0, s1, s2):
        wid = lax.axis_index("s") * 2 + lax.axis_index("c")
        bt = wid * ptok
        c0 = pltpu.async_copy(p0_hbm.at[pl.ds(bt, ptok)], pv0, s0)
        c1 = pltpu.async_copy(p1_hbm.at[pl.ds(bt, ptok)], pv1, s1)
        c2 = pltpu.async_copy(x_hbm.at[pl.ds(bt, ptok)], xv, s2)
        c0.wait()
        c1.wait()
        c2.wait()
        d0 = pltpu.async_copy(xv, xs_hbm.at[pv0], s0)
        d1 = pltpu.async_copy(xv, xs_hbm.at[pv1], s1)
        d0.wait()
        d1.wait()

    return k(x, pos0, pos1)


def _mlp_body(tile_s, exp_s, lo_s, hi_s,
              xs_ref, w1_ref, b1_ref, w2_ref, b2_ref, out_ref):
    s = pl.program_id(0)
    lo, hi = lo_s[s], hi_s[s]

    @pl.when(hi > lo)
    def _():
        xb = xs_ref[...].astype(jnp.bfloat16)
        h = jnp.dot(xb, w1_ref[0].astype(jnp.bfloat16),
                    preferred_element_type=jnp.float32)
        h = jnp.maximum(h + b1_ref[0], 0.0).astype(jnp.bfloat16)
        y = jnp.dot(h, w2_ref[0].astype(jnp.bfloat16),
                    preferred_element_type=jnp.float32)
        y = y + b2_ref[0]
        base = tile_s[s] * BMM
        rows = base + lax.broadcasted_iota(jnp.int32, (BMM, 1), 0)
        mask = (rows >= lo) & (rows < hi)
        out_ref[...] = jnp.where(mask, y, out_ref[...])


def _mlp_grouped(xs, W1, b1, W2, b2, tile_s, exp_s, lo_s, hi_s):
    grid_spec = pltpu.PrefetchScalarGridSpec(
        num_scalar_prefetch=4,
        grid=(S,),
        in_specs=[
            pl.BlockSpec((BMM, D), lambda s, t, e, lo, hi: (t[s], 0)),
            pl.BlockSpec((1, D, H), lambda s, t, e, lo, hi: (e[s], 0, 0)),
            pl.BlockSpec((1, 1, H), lambda s, t, e, lo, hi: (e[s], 0, 0)),
            pl.BlockSpec((1, H, O), lambda s, t, e, lo, hi: (e[s], 0, 0)),
            pl.BlockSpec((1, 1, O), lambda s, t, e, lo, hi: (e[s], 0, 0)),
        ],
        out_specs=pl.BlockSpec((BMM, O), lambda s, t, e, lo, hi: (t[s], 0)),
    )
    return pl.pallas_call(
        _mlp_body,
        grid_spec=grid_spec,
        out_shape=jax.ShapeDtypeStruct((SLOTS, O), jnp.float32),
    )(tile_s, exp_s, lo_s, hi_s, xs, W1, b1.reshape(E, 1, H), W2,
      b2.reshape(E, 1, O))


def _sc_combine_gather(ys, posI):
    pslot = SLOTS // NW
    mesh = plsc.VectorSubcoreMesh(core_axis_name="c", subcore_axis_name="s")

    @functools.partial(
        pl.kernel, mesh=mesh,
        out_type=jax.ShapeDtypeStruct((SLOTS, O), jnp.float32),
        scratch_types=[
            pltpu.VMEM((pslot,), jnp.int32),
            pltpu.VMEM((pslot, O), jnp.float32),
            pltpu.SemaphoreType.DMA,
        ],
    )
    def k(ys_hbm, pi_hbm, g_hbm, pvi, rows_v, sem):
        wid = lax.axis_index("s") * 2 + lax.axis_index("c")
        bs = wid * pslot
        pltpu.sync_copy(pi_hbm.at[pl.ds(bs, pslot)], pvi)
        pltpu.async_copy(ys_hbm.at[pvi], rows_v, sem).wait()
        pltpu.sync_copy(rows_v, g_hbm.at[pl.ds(bs, pslot)])

    return k(ys, posI)


def _add_body(g_ref, w_ref, o_ref):
    o_ref[...] = (w_ref[:, 0:1] * g_ref[:, :O]
                  + w_ref[:, 1:2] * g_ref[:, O:])


def _pair_add(g2, gate_w):
    return pl.pallas_call(
        _add_body,
        grid=(N // BM,),
        in_specs=[pl.BlockSpec((BM, K * O), lambda i: (i, 0)),
                  pl.BlockSpec((BM, K), lambda i: (i, 0))],
        out_specs=pl.BlockSpec((BM, O), lambda i: (i, 0)),
        out_shape=jax.ShapeDtypeStruct((N, O), jnp.float32),
    )(g2, gate_w)


def kernel(x, Wg, bg, W1, b1, W2, b2):
    gate_w, gate_i = _gating(x, Wg, bg)
    pos2d, t_s, e_s, lo_s, hi_s = _rank(gate_i.reshape(_RROWS, 128))
    posI = pos2d.reshape(SLOTS)
    posnk = pos2d.reshape(N, K)
    xs = _sc_dispatch(x, posnk[:, 0], posnk[:, 1])
    ys = _mlp_grouped(xs, W1, b1, W2, b2,
                      t_s.reshape(S), e_s.reshape(S),
                      lo_s.reshape(S), hi_s.reshape(S))
    g = _sc_combine_gather(ys, posI)
    return _pair_add(g.reshape(N, K * O), gate_w)
    posI = pos2d.reshape(SLOTS)
    posnk = pos2d.reshape(N, K)

    xs = _sc_dispatch(x, posnk[:, 0], posnk[:, 1])
    ys = _mlp_grouped(xs, W1, b1, W2, b2,
                      t_s.reshape(S), e_s.reshape(S),
                      lo_s.reshape(S), hi_s.reshape(S))

    g = _sc_combine_gather(ys, posI)
    return _pair_add(g.reshape(N, K, O), gate_w)


# speedup vs baseline: 1.1245x; 1.0140x over previous
import functools

import jax
import jax.numpy as jnp
from jax import lax
from jax.experimental import pallas as pl
from jax.experimental.pallas import tpu as pltpu
from jax.experimental.pallas import tpu_sc as plsc

N, D, H, O, E, K = 2048, 768, 768, 768, 8, 2
BM = 256
BMM = 512
T = (N * K) // BMM
S = T + E - 1
NW = 32
SLOTS = N * K


def _gating_body(x_ref, wg_ref, bg_ref, ww_ref, i_ref):
    xv = x_ref[...]
    logits = jnp.dot(xv, wg_ref[...],
                     preferred_element_type=jnp.float32) + bg_ref[...]
    cols = lax.broadcasted_iota(jnp.int32, logits.shape, 1)
    m1 = jnp.max(logits, axis=-1, keepdims=True)
    i1 = jnp.min(jnp.where(logits == m1, cols, E), axis=-1, keepdims=True)
    l2 = jnp.where(cols == i1, -jnp.inf, logits)
    m2 = jnp.max(l2, axis=-1, keepdims=True)
    i2 = jnp.min(jnp.where(l2 == m2, cols, E), axis=-1, keepdims=True)
    e2 = jnp.exp(m2 - m1)
    denom = 1.0 + e2
    ww_ref[...] = jnp.concatenate([1.0 / denom, e2 / denom], axis=-1)
    i_ref[...] = jnp.concatenate([i1, i2], axis=-1)


def _gating(x, Wg, bg):
    return pl.pallas_call(
        _gating_body,
        grid=(N // BM,),
        in_specs=[
            pl.BlockSpec((BM, D), lambda i: (i, 0)),
            pl.BlockSpec((D, E), lambda i: (0, 0)),
            pl.BlockSpec((1, E), lambda i: (0, 0)),
        ],
        out_specs=[
            pl.BlockSpec((BM, K), lambda i: (i, 0)),
            pl.BlockSpec((BM, K), lambda i: (i, 0)),
        ],
        out_shape=[
            jax.ShapeDtypeStruct((N, K), jnp.float32),
            jax.ShapeDtypeStruct((N, K), jnp.int32),
        ],
    )(x, Wg, bg.reshape(1, E))


_RROWS = SLOTS // 128


def _rank_body(e_ref, pos_ref, t_ref, eo_ref, lo_ref, hi_ref):
    f32 = jnp.float32
    ef = e_ref[...]
    lane_tri = (lax.broadcasted_iota(jnp.int32, (128, 128), 0)
                <= lax.broadcasted_iota(jnp.int32, (128, 128), 1)
                ).astype(f32)
    row_strict = (lax.broadcasted_iota(jnp.int32, (_RROWS, _RROWS), 1)
                  < lax.broadcasted_iota(jnp.int32, (_RROWS, _RROWS), 0)
                  ).astype(f32)

    pos = jnp.zeros((_RROWS, 128), f32)
    off = 0.0
    offs_e, offs_i = [], []
    for e in range(E):
        m = (ef == e).astype(f32)
        lane_pref = jnp.dot(m, lane_tri, preferred_element_type=f32)
        rt = jnp.sum(m, axis=1, keepdims=True)
        row_pe = jnp.dot(row_strict, rt, preferred_element_type=f32)
        rank0 = lane_pref + row_pe - 1.0
        offs_e.append(off)
        pos = pos + m * (off + rank0)
        off = off + jnp.sum(rt)
        offs_i.append(off)
    pos_ref[...] = pos.astype(jnp.int32)

    off_e_row = jnp.stack(offs_e).reshape(1, E)
    off_i_row = jnp.stack(offs_i).reshape(1, E)
    off_e_col = jnp.stack(offs_e).reshape(E, 1)
    off_i_col = jnp.stack(offs_i).reshape(E, 1)

    tcol = lax.broadcasted_iota(jnp.int32, (T, 1), 0).astype(f32)
    incl_a = (jnp.minimum((tcol + 1.0) * BMM, off_i_row)
              > jnp.maximum(tcol * BMM, off_e_row)).astype(f32)
    nsteps = jnp.sum(incl_a, axis=1, keepdims=True)
    l16s = (lax.broadcasted_iota(jnp.int32, (T, T), 1)
            < lax.broadcasted_iota(jnp.int32, (T, T), 0)).astype(f32)
    csteps = jnp.dot(l16s, nsteps, preferred_element_type=f32)

    s_row = lax.broadcasted_iota(jnp.int32, (1, S), 1).astype(f32)
    t_s = jnp.sum((csteps <= s_row).astype(f32), axis=0, keepdims=True) - 1.0
    onehot_t = (lax.broadcasted_iota(jnp.int32, (T, S), 0).astype(f32) == t_s).astype(f32)
    cst_s = jnp.sum(onehot_t * csteps, axis=0, keepdims=True)
    nst_s = jnp.sum(onehot_t * nsteps, axis=0, keepdims=True)
    j = jnp.minimum(s_row - cst_s, jnp.maximum(nst_s - 1.0, 0.0))

    trow = lax.broadcasted_iota(jnp.int32, (1, T), 1).astype(f32)
    incl_b = (jnp.minimum((trow + 1.0) * BMM, off_i_col)
              > jnp.maximum(trow * BMM, off_e_col)).astype(f32)
    l8i = (lax.broadcasted_iota(jnp.int32, (E, E), 1)
           <= lax.broadcasted_iota(jnp.int32, (E, E), 0)).astype(f32)
    cc_b = jnp.dot(l8i, incl_b, preferred_element_type=f32)
    cc_sel = jnp.dot(cc_b, onehot_t, preferred_element_type=f32)
    e_s = jnp.sum((cc_sel <= j).astype(f32), axis=0, keepdims=True)
    e_s = jnp.minimum(e_s, float(E - 1))
    onehot_e = (lax.broadcasted_iota(jnp.int32, (E, S), 0).astype(f32) == e_s).astype(f32)
    osel_e = jnp.sum(onehot_e * off_e_col, axis=0, keepdims=True)
    osel_i = jnp.sum(onehot_e * off_i_col, axis=0, keepdims=True)
    lo_s = jnp.maximum(t_s * BMM, osel_e)
    hi_s = jnp.minimum((t_s + 1.0) * BMM, osel_i)
    total_steps = jnp.sum(nsteps)
    padm = s_row >= total_steps
    lo_s = jnp.where(padm, 0.0, lo_s)
    hi_s = jnp.where(padm, 0.0, hi_s)

    t_ref[...] = t_s.astype(jnp.int32)
    eo_ref[...] = e_s.astype(jnp.int32)
    lo_ref[...] = lo_s.astype(jnp.int32)
    hi_ref[...] = hi_s.astype(jnp.int32)


def _rank(e2d):
    return pl.pallas_call(
        _rank_body,
        out_shape=[
            jax.ShapeDtypeStruct((_RROWS, 128), jnp.int32),
            jax.ShapeDtypeStruct((1, S), jnp.int32),
            jax.ShapeDtypeStruct((1, S), jnp.int32),
            jax.ShapeDtypeStruct((1, S), jnp.int32),
            jax.ShapeDtypeStruct((1, S), jnp.int32),
        ],
    )(e2d)


def _sc_dispatch(x, pos0, pos1):
    ptok = N // NW
    pslot = SLOTS // NW
    mesh = plsc.VectorSubcoreMesh(core_axis_name="c", subcore_axis_name="s")

    @functools.partial(
        pl.kernel, mesh=mesh,
        out_type=jax.ShapeDtypeStruct((SLOTS, D), jnp.float32),
        scratch_types=[
            pltpu.VMEM((ptok, D), jnp.float32),
            pltpu.VMEM((ptok,), jnp.int32),
            pltpu.VMEM((ptok,), jnp.int32),
            pltpu.SemaphoreType.DMA,
            pltpu.SemaphoreType.DMA,
            pltpu.SemaphoreType.DMA,
        ],
    )
    def k(x_hbm, p0_hbm, p1_hbm, xs_hbm, xv, pv0, pv1, s0, s1, s2):
        wid = lax.axis_index("s") * 2 + lax.axis_index("c")
        bt = wid * ptok
        c0 = pltpu.async_copy(p0_hbm.at[pl.ds(bt, ptok)], pv0, s0)
        c1 = pltpu.async_copy(p1_hbm.at[pl.ds(bt, ptok)], pv1, s1)
        c2 = pltpu.async_copy(x_hbm.at[pl.ds(bt, ptok)], xv, s2)
        c0.wait()
        c1.wait()
        c2.wait()
        d0 = pltpu.async_copy(xv, xs_hbm.at[pv0], s0)
        d1 = pltpu.async_copy(xv, xs_hbm.at[pv1], s1)
        d0.wait()
        d1.wait()

    return k(x, pos0, pos1)


def _mlp_body(tile_s, exp_s, lo_s, hi_s,
              xs_ref, w1_ref, b1_ref, w2_ref, b2_ref, out_ref):
    s = pl.program_id(0)
    lo, hi = lo_s[s], hi_s[s]

    @pl.when(hi > lo)
    def _():
        xb = xs_ref[...].astype(jnp.bfloat16)
        h = jnp.dot(xb, w1_ref[0].astype(jnp.bfloat16),
                    preferred_element_type=jnp.float32)
        h = jnp.maximum(h + b1_ref[0], 0.0).astype(jnp.bfloat16)
        y = jnp.dot(h, w2_ref[0].astype(jnp.bfloat16),
                    preferred_element_type=jnp.float32)
        y = y + b2_ref[0]
        base = tile_s[s] * BMM
        rows = base + lax.broadcasted_iota(jnp.int32, (BMM, 1), 0)
        mask = (rows >= lo) & (rows < hi)
        out_ref[...] = jnp.where(mask, y, out_ref[...])


def _mlp_grouped(xs, W1, b1, W2, b2, tile_s, exp_s, lo_s, hi_s):
    grid_spec = pltpu.PrefetchScalarGridSpec(
        num_scalar_prefetch=4,
        grid=(S,),
        in_specs=[
            pl.BlockSpec((BMM, D), lambda s, t, e, lo, hi: (t[s], 0)),
            pl.BlockSpec((1, D, H), lambda s, t, e, lo, hi: (e[s], 0, 0)),
            pl.BlockSpec((1, 1, H), lambda s, t, e, lo, hi: (e[s], 0, 0)),
            pl.BlockSpec((1, H, O), lambda s, t, e, lo, hi: (e[s], 0, 0)),
            pl.BlockSpec((1, 1, O), lambda s, t, e, lo, hi: (e[s], 0, 0)),
        ],
        out_specs=pl.BlockSpec((BMM, O), lambda s, t, e, lo, hi: (t[s], 0)),
    )
    return pl.pallas_call(
        _mlp_body,
        grid_spec=grid_spec,
        out_shape=jax.ShapeDtypeStruct((SLOTS, O), jnp.float32),
    )(tile_s, exp_s, lo_s, hi_s, xs, W1, b1.reshape(E, 1, H), W2,
      b2.reshape(E, 1, O))


def _sc_combine_gather(ys, posI):
    pslot = SLOTS // NW
    mesh = plsc.VectorSubcoreMesh(core_axis_name="c", subcore_axis_name="s")

    @functools.partial(
        pl.kernel, mesh=mesh,
        out_type=jax.ShapeDtypeStruct((SLOTS, O), jnp.float32),
        scratch_types=[
            pltpu.VMEM((pslot,), jnp.int32),
            pltpu.VMEM((pslot, O), jnp.float32),
            pltpu.SemaphoreType.DMA,
        ],
    )
    def k(ys_hbm, pi_hbm, g_hbm, pvi, rows_v, sem):
        wid = lax.axis_index("s") * 2 + lax.axis_index("c")
        bs = wid * pslot
        pltpu.sync_copy(pi_hbm.at[pl.ds(bs, pslot)], pvi)
        pltpu.async_copy(ys_hbm.at[pvi], rows_v, sem).wait()
        pltpu.sync_copy(rows_v, g_hbm.at[pl.ds(bs, pslot)])

    return k(ys, posI)


def _add_body(g_ref, w_ref, o_ref):
    o_ref[...] = (w_ref[:, 0:1] * g_ref[:, :O]
                  + w_ref[:, 1:2] * g_ref[:, O:])


def _pair_add(g2, gate_w):
    return pl.pallas_call(
        _add_body,
        grid=(N // BMM,),
        in_specs=[pl.BlockSpec((BMM, K * O), lambda i: (i, 0)),
                  pl.BlockSpec((BMM, K), lambda i: (i, 0))],
        out_specs=pl.BlockSpec((BMM, O), lambda i: (i, 0)),
        out_shape=jax.ShapeDtypeStruct((N, O), jnp.float32),
    )(g2, gate_w)


def kernel(x, Wg, bg, W1, b1, W2, b2):
    gate_w, gate_i = _gating(x, Wg, bg)
    pos2d, t_s, e_s, lo_s, hi_s = _rank(gate_i.reshape(_RROWS, 128))
    posI = pos2d.reshape(SLOTS)
    posnk = pos2d.reshape(N, K)
    xs = _sc_dispatch(x, posnk[:, 0], posnk[:, 1])
    ys = _mlp_grouped(xs, W1, b1, W2, b2,
                      t_s.reshape(S), e_s.reshape(S),
                      lo_s.reshape(S), hi_s.reshape(S))
    g = _sc_combine_gather(ys, posI)
    return _pair_add(g.reshape(N, K * O), gate_w)
    posI = pos2d.reshape(SLOTS)
    posnk = pos2d.reshape(N, K)

    xs = _sc_dispatch(x, posnk[:, 0], posnk[:, 1])
    ys = _mlp_grouped(xs, W1, b1, W2, b2,
                      t_s.reshape(S), e_s.reshape(S),
                      lo_s.reshape(S), hi_s.reshape(S))

    g = _sc_combine_gather(ys, posI)
    return _pair_add(g.reshape(N, K, O), gate_w)


# speedup vs baseline: 1.1543x; 1.0265x over previous
import functools

import jax
import jax.numpy as jnp
from jax import lax
from jax.experimental import pallas as pl
from jax.experimental.pallas import tpu as pltpu
from jax.experimental.pallas import tpu_sc as plsc

N, D, H, O, E, K = 2048, 768, 768, 768, 8, 2
BM = 256
BMM = 512
T = (N * K) // BMM
S = T + E - 1
NW = 32
SLOTS = N * K


def _gating_body(x_ref, wg_ref, bg_ref, ww_ref, i_ref):
    xv = x_ref[...]
    logits = jnp.dot(xv, wg_ref[...],
                     preferred_element_type=jnp.float32) + bg_ref[...]
    cols = lax.broadcasted_iota(jnp.int32, logits.shape, 1)
    m1 = jnp.max(logits, axis=-1, keepdims=True)
    i1 = jnp.min(jnp.where(logits == m1, cols, E), axis=-1, keepdims=True)
    l2 = jnp.where(cols == i1, -jnp.inf, logits)
    m2 = jnp.max(l2, axis=-1, keepdims=True)
    i2 = jnp.min(jnp.where(l2 == m2, cols, E), axis=-1, keepdims=True)
    e2 = jnp.exp(m2 - m1)
    denom = 1.0 + e2
    ww_ref[...] = jnp.concatenate([1.0 / denom, e2 / denom], axis=-1)
    i_ref[...] = jnp.concatenate([i1, i2], axis=-1)


def _gating(x, Wg, bg):
    return pl.pallas_call(
        _gating_body,
        grid=(N // BMM,),
        in_specs=[
            pl.BlockSpec((BMM, D), lambda i: (i, 0)),
            pl.BlockSpec((D, E), lambda i: (0, 0)),
            pl.BlockSpec((1, E), lambda i: (0, 0)),
        ],
        out_specs=[
            pl.BlockSpec((BMM, K), lambda i: (i, 0)),
            pl.BlockSpec((BMM, K), lambda i: (i, 0)),
        ],
        out_shape=[
            jax.ShapeDtypeStruct((N, K), jnp.float32),
            jax.ShapeDtypeStruct((N, K), jnp.int32),
        ],
    )(x, Wg, bg.reshape(1, E))


_RROWS = SLOTS // 128


def _rank_body(e_ref, pos_ref, t_ref, eo_ref, lo_ref, hi_ref):
    f32 = jnp.float32
    ef = e_ref[...]
    lane_tri = (lax.broadcasted_iota(jnp.int32, (128, 128), 0)
                <= lax.broadcasted_iota(jnp.int32, (128, 128), 1)
                ).astype(f32)
    row_strict = (lax.broadcasted_iota(jnp.int32, (_RROWS, _RROWS), 1)
                  < lax.broadcasted_iota(jnp.int32, (_RROWS, _RROWS), 0)
                  ).astype(f32)

    pos = jnp.zeros((_RROWS, 128), f32)
    off = 0.0
    offs_e, offs_i = [], []
    for e in range(E):
        m = (ef == e).astype(f32)
        lane_pref = jnp.dot(m, lane_tri, preferred_element_type=f32)
        rt = jnp.sum(m, axis=1, keepdims=True)
        row_pe = jnp.dot(row_strict, rt, preferred_element_type=f32)
        rank0 = lane_pref + row_pe - 1.0
        offs_e.append(off)
        pos = pos + m * (off + rank0)
        off = off + jnp.sum(rt)
        offs_i.append(off)
    pos_ref[...] = pos.astype(jnp.int32)

    off_e_row = jnp.stack(offs_e).reshape(1, E)
    off_i_row = jnp.stack(offs_i).reshape(1, E)
    off_e_col = jnp.stack(offs_e).reshape(E, 1)
    off_i_col = jnp.stack(offs_i).reshape(E, 1)

    tcol = lax.broadcasted_iota(jnp.int32, (T, 1), 0).astype(f32)
    incl_a = (jnp.minimum((tcol + 1.0) * BMM, off_i_row)
              > jnp.maximum(tcol * BMM, off_e_row)).astype(f32)
    nsteps = jnp.sum(incl_a, axis=1, keepdims=True)
    l16s = (lax.broadcasted_iota(jnp.int32, (T, T), 1)
            < lax.broadcasted_iota(jnp.int32, (T, T), 0)).astype(f32)
    csteps = jnp.dot(l16s, nsteps, preferred_element_type=f32)

    s_row = lax.broadcasted_iota(jnp.int32, (1, S), 1).astype(f32)
    t_s = jnp.sum((csteps <= s_row).astype(f32), axis=0, keepdims=True) - 1.0
    onehot_t = (lax.broadcasted_iota(jnp.int32, (T, S), 0).astype(f32) == t_s).astype(f32)
    cst_s = jnp.sum(onehot_t * csteps, axis=0, keepdims=True)
    nst_s = jnp.sum(onehot_t * nsteps, axis=0, keepdims=True)
    j = jnp.minimum(s_row - cst_s, jnp.maximum(nst_s - 1.0, 0.0))

    trow = lax.broadcasted_iota(jnp.int32, (1, T), 1).astype(f32)
    incl_b = (jnp.minimum((trow + 1.0) * BMM, off_i_col)
              > jnp.maximum(trow * BMM, off_e_col)).astype(f32)
    l8i = (lax.broadcasted_iota(jnp.int32, (E, E), 1)
           <= lax.broadcasted_iota(jnp.int32, (E, E), 0)).astype(f32)
    cc_b = jnp.dot(l8i, incl_b, preferred_element_type=f32)
    cc_sel = jnp.dot(cc_b, onehot_t, preferred_element_type=f32)
    e_s = jnp.sum((cc_sel <= j).astype(f32), axis=0, keepdims=True)
    e_s = jnp.minimum(e_s, float(E - 1))
    onehot_e = (lax.broadcasted_iota(jnp.int32, (E, S), 0).astype(f32) == e_s).astype(f32)
    osel_e = jnp.sum(onehot_e * off_e_col, axis=0, keepdims=True)
    osel_i = jnp.sum(onehot_e * off_i_col, axis=0, keepdims=True)
    lo_s = jnp.maximum(t_s * BMM, osel_e)
    hi_s = jnp.minimum((t_s + 1.0) * BMM, osel_i)
    total_steps = jnp.sum(nsteps)
    padm = s_row >= total_steps
    lo_s = jnp.where(padm, 0.0, lo_s)
    hi_s = jnp.where(padm, 0.0, hi_s)

    t_ref[...] = t_s.astype(jnp.int32)
    eo_ref[...] = e_s.astype(jnp.int32)
    lo_ref[...] = lo_s.astype(jnp.int32)
    hi_ref[...] = hi_s.astype(jnp.int32)


def _rank(e2d):
    return pl.pallas_call(
        _rank_body,
        out_shape=[
            jax.ShapeDtypeStruct((_RROWS, 128), jnp.int32),
            jax.ShapeDtypeStruct((1, S), jnp.int32),
            jax.ShapeDtypeStruct((1, S), jnp.int32),
            jax.ShapeDtypeStruct((1, S), jnp.int32),
            jax.ShapeDtypeStruct((1, S), jnp.int32),
        ],
    )(e2d)


def _sc_dispatch(x, pos0, pos1):
    ptok = N // NW
    pslot = SLOTS // NW
    mesh = plsc.VectorSubcoreMesh(core_axis_name="c", subcore_axis_name="s")

    @functools.partial(
        pl.kernel, mesh=mesh,
        out_type=jax.ShapeDtypeStruct((SLOTS, D), jnp.float32),
        scratch_types=[
            pltpu.VMEM((ptok, D), jnp.float32),
            pltpu.VMEM((ptok,), jnp.int32),
            pltpu.VMEM((ptok,), jnp.int32),
            pltpu.SemaphoreType.DMA,
            pltpu.SemaphoreType.DMA,
            pltpu.SemaphoreType.DMA,
        ],
    )
    def k(x_hbm, p0_hbm, p1_hbm, xs_hbm, xv, pv0, pv1, s0, s1, s2):
        wid = lax.axis_index("s") * 2 + lax.axis_index("c")
        bt = wid * ptok
        c0 = pltpu.async_copy(p0_hbm.at[pl.ds(bt, ptok)], pv0, s0)
        c1 = pltpu.async_copy(p1_hbm.at[pl.ds(bt, ptok)], pv1, s1)
        c2 = pltpu.async_copy(x_hbm.at[pl.ds(bt, ptok)], xv, s2)
        c0.wait()
        c1.wait()
        c2.wait()
        d0 = pltpu.async_copy(xv, xs_hbm.at[pv0], s0)
        d1 = pltpu.async_copy(xv, xs_hbm.at[pv1], s1)
        d0.wait()
        d1.wait()

    return k(x, pos0, pos1)


def _mlp_body(tile_s, exp_s, lo_s, hi_s,
              xs_ref, w1_ref, b1_ref, w2_ref, b2_ref, out_ref):
    s = pl.program_id(0)
    lo, hi = lo_s[s], hi_s[s]

    @pl.when(hi > lo)
    def _():
        xb = xs_ref[...].astype(jnp.bfloat16)
        h = jnp.dot(xb, w1_ref[0].astype(jnp.bfloat16),
                    preferred_element_type=jnp.float32)
        h = jnp.maximum(h + b1_ref[0], 0.0).astype(jnp.bfloat16)
        y = jnp.dot(h, w2_ref[0].astype(jnp.bfloat16),
                    preferred_element_type=jnp.float32)
        y = y + b2_ref[0]
        base = tile_s[s] * BMM
        rows = base + lax.broadcasted_iota(jnp.int32, (BMM, 1), 0)
        mask = (rows >= lo) & (rows < hi)
        out_ref[...] = jnp.where(mask, y, out_ref[...])


def _mlp_grouped(xs, W1, b1, W2, b2, tile_s, exp_s, lo_s, hi_s):
    grid_spec = pltpu.PrefetchScalarGridSpec(
        num_scalar_prefetch=4,
        grid=(S,),
        in_specs=[
            pl.BlockSpec((BMM, D), lambda s, t, e, lo, hi: (t[s], 0)),
            pl.BlockSpec((1, D, H), lambda s, t, e, lo, hi: (e[s], 0, 0)),
            pl.BlockSpec((1, 1, H), lambda s, t, e, lo, hi: (e[s], 0, 0)),
            pl.BlockSpec((1, H, O), lambda s, t, e, lo, hi: (e[s], 0, 0)),
            pl.BlockSpec((1, 1, O), lambda s, t, e, lo, hi: (e[s], 0, 0)),
        ],
        out_specs=pl.BlockSpec((BMM, O), lambda s, t, e, lo, hi: (t[s], 0)),
    )
    return pl.pallas_call(
        _mlp_body,
        grid_spec=grid_spec,
        out_shape=jax.ShapeDtypeStruct((SLOTS, O), jnp.float32),
    )(tile_s, exp_s, lo_s, hi_s, xs, W1, b1.reshape(E, 1, H), W2,
      b2.reshape(E, 1, O))


def _sc_combine_gather(ys, posI):
    pslot = SLOTS // NW
    mesh = plsc.VectorSubcoreMesh(core_axis_name="c", subcore_axis_name="s")

    @functools.partial(
        pl.kernel, mesh=mesh,
        out_type=jax.ShapeDtypeStruct((SLOTS, O), jnp.float32),
        scratch_types=[
            pltpu.VMEM((pslot,), jnp.int32),
            pltpu.VMEM((pslot, O), jnp.float32),
            pltpu.SemaphoreType.DMA,
        ],
    )
    def k(ys_hbm, pi_hbm, g_hbm, pvi, rows_v, sem):
        wid = lax.axis_index("s") * 2 + lax.axis_index("c")
        bs = wid * pslot
        pltpu.sync_copy(pi_hbm.at[pl.ds(bs, pslot)], pvi)
        pltpu.async_copy(ys_hbm.at[pvi], rows_v, sem).wait()
        pltpu.sync_copy(rows_v, g_hbm.at[pl.ds(bs, pslot)])

    return k(ys, posI)


def _add_body(g_ref, w_ref, o_ref):
    o_ref[...] = (w_ref[:, 0:1] * g_ref[:, :O]
                  + w_ref[:, 1:2] * g_ref[:, O:])


def _pair_add(g2, gate_w):
    return pl.pallas_call(
        _add_body,
        grid=(N // BMM,),
        in_specs=[pl.BlockSpec((BMM, K * O), lambda i: (i, 0)),
                  pl.BlockSpec((BMM, K), lambda i: (i, 0))],
        out_specs=pl.BlockSpec((BMM, O), lambda i: (i, 0)),
        out_shape=jax.ShapeDtypeStruct((N, O), jnp.float32),
    )(g2, gate_w)


def kernel(x, Wg, bg, W1, b1, W2, b2):
    gate_w, gate_i = _gating(x, Wg, bg)
    pos2d, t_s, e_s, lo_s, hi_s = _rank(gate_i.reshape(_RROWS, 128))
    posI = pos2d.reshape(SLOTS)
    posnk = pos2d.reshape(N, K)
    xs = _sc_dispatch(x, posnk[:, 0], posnk[:, 1])
    ys = _mlp_grouped(xs, W1, b1, W2, b2,
                      t_s.reshape(S), e_s.reshape(S),
                      lo_s.reshape(S), hi_s.reshape(S))
    g = _sc_combine_gather(ys, posI)
    return _pair_add(g.reshape(N, K * O), gate_w)
    posI = pos2d.reshape(SLOTS)
    posnk = pos2d.reshape(N, K)

    xs = _sc_dispatch(x, posnk[:, 0], posnk[:, 1])
    ys = _mlp_grouped(xs, W1, b1, W2, b2,
                      t_s.reshape(S), e_s.reshape(S),
                      lo_s.reshape(S), hi_s.reshape(S))

    g = _sc_combine_gather(ys, posI)
    return _pair_add(g.reshape(N, K, O), gate_w)
